# baseline (speedup 1.0000x reference)
import functools

import jax
import jax.numpy as jnp
import numpy as np
from jax.experimental import pallas as pl
from jax.experimental.pallas import tpu as pltpu

_KY, _KX = np.meshgrid(np.arange(5), np.arange(5), indexing="ij")
_TI = (_KY // 2) * 3 + (_KX // 2)
_QI = (_KY % 2) * 2 + (_KX % 2)


def _conv1_slab(w1):
    wt1 = w1.reshape(9, 4, 4, 16)[:, :, 0, :][_TI, _QI]
    v, x, s, dy, dx, w2 = np.ix_(np.arange(9), np.arange(32), np.arange(2),
                                 np.arange(2), np.arange(2), np.arange(14))
    ky = v - 2 * s - dy
    kx = x - 2 * w2 - dx
    mask = (ky >= 0) & (ky < 5) & (kx >= 0) & (kx < 5)
    slab = wt1[np.clip(ky, 0, 4), np.clip(kx, 0, 4)]
    slab = jnp.where(jnp.asarray(mask[..., None]), slab, 0.0)
    return slab.reshape(288, 1792).astype(jnp.bfloat16)


def _conv2_slab(w2):
    wt2 = w2.reshape(9, 4, 16, 4, 32)[:, :, :, 0, :][_TI, _QI]
    u, w, dy, dx, j2 = np.ix_(np.arange(7), np.arange(14), np.arange(2),
                              np.arange(2), np.arange(7))
    ky = u - dy
    kx = w - 2 * j2 - dx + 2
    mask = (ky >= 0) & (ky < 5) & (kx >= 0) & (kx < 5)
    slab = wt2[np.clip(ky, 0, 4), np.clip(kx, 0, 4)]
    slab = jnp.where(jnp.asarray(mask[..., None, None]), slab, 0.0)
    slab = slab.transpose(0, 1, 5, 2, 3, 6, 4)
    return slab.reshape(1568, 896).astype(jnp.bfloat16)


_PF = np.zeros((1568, 1568), np.float32)
for _i in range(7):
    for _co in range(32):
        for _j in range(7):
            _PF[_i * 224 + _co * 7 + _j, _co * 49 + _i * 7 + _j] = 1.0


def _fused_kernel(xc_ref, w1_ref, b1_ref, w2_ref, b2_ref, wl_ref, bo_ref,
                  pf_ref, logits_ref, flat_ref, *, tb):
    acc1 = jnp.dot(xc_ref[...].reshape(tb * 7, 288), w1_ref[...],
                   preferred_element_type=jnp.float32)
    ps = []
    for s in (0, 1):
        a = acc1[:, s * 896:(s + 1) * 896]
        m = jnp.maximum(jnp.maximum(a[:, 0:224], a[:, 224:448]),
                        jnp.maximum(a[:, 448:672], a[:, 672:896]))
        m = jnp.maximum(m + b1_ref[...], 0.0).astype(jnp.bfloat16)
        ps.append(m.reshape(tb, 7, 224))
    pe = jnp.pad(ps[0], ((0, 0), (1, 2), (0, 0)))
    po = jnp.pad(ps[1], ((0, 0), (1, 2), (0, 0)))
    xcat2 = jnp.concatenate(
        [(pe if u % 2 == 0 else po)[:, u // 2:u // 2 + 7, :]
         for u in range(7)],
        axis=2).reshape(tb * 7, 1568)
    acc2 = jnp.dot(xcat2, w2_ref[...],
                   preferred_element_type=jnp.float32)
    q = jnp.maximum(jnp.maximum(acc2[:, 0:224], acc2[:, 224:448]),
                    jnp.maximum(acc2[:, 448:672], acc2[:, 672:896]))
    q = jnp.maximum(q + b2_ref[...], 0.0)
    q3 = q.reshape(tb, 7, 224)

    qcat = jnp.concatenate([q3[:, i, :] for i in range(7)], axis=1)
    qb = qcat.astype(jnp.bfloat16)
    logits_ref[...] = (
        jnp.dot(qb, wl_ref[...],
                preferred_element_type=jnp.float32) + bo_ref[...])

    flat_ref[...] = jnp.dot(qb, pf_ref[...],
                            preferred_element_type=jnp.float32)


def _forward(x_nchw, w1, b1, w2, b2, w_out, b_out, *, tb=128):
    B = x_nchw.shape[0]

    xp = jnp.pad(x_nchw.reshape(B, 28, 28), ((0, 0), (2, 6), (2, 2)))
    xc = (jnp.stack([xp[:, 4 * g:4 * g + 9, :] for g in range(7)], axis=1)
          .reshape(B, 7, 288).astype(jnp.bfloat16))

    w1s = _conv1_slab(w1)
    w2s = _conv2_slab(w2)
    b1t = jnp.tile(b1.reshape(16), (14,)).reshape(1, 224)
    b2t = jnp.repeat(b2.reshape(32), 7).reshape(1, 224)
    wl = (w_out.reshape(32, 7, 7, 128).transpose(1, 0, 2, 3)
          .reshape(1568, 128).astype(jnp.bfloat16))

    pf = jnp.asarray(_PF, jnp.bfloat16)

    flops = 2 * B * 7 * (288 * 1792 + 1568 * 896) + 4 * B * 1568 * 128
    bytes_accessed = 2 * B * 7 * 288 + 4 * B * (128 + 1568)

    logits_pad, flat = pl.pallas_call(
        functools.partial(_fused_kernel, tb=tb),
        out_shape=[jax.ShapeDtypeStruct((B, 128), jnp.float32),
                   jax.ShapeDtypeStruct((B, 1568), jnp.float32)],
        grid=(B // tb,),
        in_specs=[pl.BlockSpec((tb, 7, 288), lambda i: (i, 0, 0)),
                  pl.BlockSpec((288, 1792), lambda i: (0, 0)),
                  pl.BlockSpec((1, 224), lambda i: (0, 0)),
                  pl.BlockSpec((1568, 896), lambda i: (0, 0)),
                  pl.BlockSpec((1, 224), lambda i: (0, 0)),
                  pl.BlockSpec((1568, 128), lambda i: (0, 0)),
                  pl.BlockSpec((1, 128), lambda i: (0, 0)),
                  pl.BlockSpec((1568, 1568), lambda i: (0, 0))],
        out_specs=[pl.BlockSpec((tb, 128), lambda i: (i, 0)),
                   pl.BlockSpec((tb, 1568), lambda i: (i, 0))],
        compiler_params=pltpu.CompilerParams(
            dimension_semantics=("parallel",)),
        cost_estimate=pl.CostEstimate(flops=flops, transcendentals=0,
                                      bytes_accessed=bytes_accessed),
    )(xc, w1s, b1t, w2s, b2t, wl, b_out.astype(jnp.float32), pf)

    return logits_pad[:, :10], flat


def kernel(x_nchw, w1, b1, w2, b2, w_out, b_out):
    return _forward(x_nchw, w1, b1, w2, b2, w_out, b_out)


# speedup vs baseline: 1.3030x; 1.3030x over previous
import functools

import jax
import jax.numpy as jnp
import numpy as np
from jax.experimental import pallas as pl
from jax.experimental.pallas import tpu as pltpu

_KY, _KX = np.meshgrid(np.arange(5), np.arange(5), indexing="ij")
_TI = (_KY // 2) * 3 + (_KX // 2)
_QI = (_KY % 2) * 2 + (_KX % 2)


def _conv1_slab(w1):
    wt1 = w1.reshape(9, 4, 4, 16)[:, :, 0, :][_TI, _QI]
    v, x, s, dy, dx, w2 = np.ix_(np.arange(9), np.arange(32), np.arange(2),
                                 np.arange(2), np.arange(2), np.arange(14))
    ky = v - 2 * s - dy
    kx = x - 2 * w2 - dx
    mask = (ky >= 0) & (ky < 5) & (kx >= 0) & (kx < 5)
    slab = wt1[np.clip(ky, 0, 4), np.clip(kx, 0, 4)]
    slab = jnp.where(jnp.asarray(mask[..., None]), slab, 0.0)
    return slab.reshape(288, 1792).astype(jnp.bfloat16)


def _conv2_slab(w2):
    wt2 = w2.reshape(9, 4, 16, 4, 32)[:, :, :, 0, :][_TI, _QI]
    u, w, dy, dx, j2 = np.ix_(np.arange(7), np.arange(14), np.arange(2),
                              np.arange(2), np.arange(7))
    ky = u - dy
    kx = w - 2 * j2 - dx + 2
    mask = (ky >= 0) & (ky < 5) & (kx >= 0) & (kx < 5)
    slab = wt2[np.clip(ky, 0, 4), np.clip(kx, 0, 4)]
    slab = jnp.where(jnp.asarray(mask[..., None, None]), slab, 0.0)
    slab = slab.transpose(0, 1, 5, 2, 3, 6, 4)
    return slab.reshape(1568, 896).astype(jnp.bfloat16)


_PF = np.zeros((1568, 1568), np.float32)
for _i in range(7):
    for _co in range(32):
        for _j in range(7):
            _PF[_i * 224 + _co * 7 + _j, _co * 49 + _i * 7 + _j] = 1.0


def _fused_kernel(xq_ref, w1_ref, b1_ref, w2_ref, b2_ref, wl_ref, bo_ref,
                  pf_ref, logits_ref, flat_ref, *, tb):
    xq = xq_ref[...]
    xcat1 = jnp.concatenate(
        [xq[:, v % 4, v // 4:v // 4 + 7, :] for v in range(9)],
        axis=2).reshape(tb * 7, 288)
    acc1 = jnp.dot(xcat1, w1_ref[...],
                   preferred_element_type=jnp.float32)
    ps = []
    for s in (0, 1):
        a = acc1[:, s * 896:(s + 1) * 896]
        m = jnp.maximum(jnp.maximum(a[:, 0:224], a[:, 224:448]),
                        jnp.maximum(a[:, 448:672], a[:, 672:896]))
        m = jnp.maximum(m + b1_ref[...], 0.0).astype(jnp.bfloat16)
        ps.append(m.reshape(tb, 7, 224))
    pe = jnp.pad(ps[0], ((0, 0), (1, 2), (0, 0)))
    po = jnp.pad(ps[1], ((0, 0), (1, 2), (0, 0)))
    xcat2 = jnp.concatenate(
        [(pe if u % 2 == 0 else po)[:, u // 2:u // 2 + 7, :]
         for u in range(7)],
        axis=2).reshape(tb * 7, 1568)
    acc2 = jnp.dot(xcat2, w2_ref[...],
                   preferred_element_type=jnp.float32)
    q = jnp.maximum(jnp.maximum(acc2[:, 0:224], acc2[:, 224:448]),
                    jnp.maximum(acc2[:, 448:672], acc2[:, 672:896]))
    q = jnp.maximum(q + b2_ref[...], 0.0)
    q3 = q.reshape(tb, 7, 224)

    qcat = jnp.concatenate([q3[:, i, :] for i in range(7)], axis=1)
    qb = qcat.astype(jnp.bfloat16)
    logits_ref[...] = (
        jnp.dot(qb, wl_ref[...],
                preferred_element_type=jnp.float32) + bo_ref[...])

    flat_ref[...] = jnp.dot(qb, pf_ref[...],
                            preferred_element_type=jnp.float32)


def _forward(x_nchw, w1, b1, w2, b2, w_out, b_out, *, tb=128):
    B = x_nchw.shape[0]

    xp = jnp.pad(x_nchw.reshape(B, 28, 28), ((0, 0), (2, 6), (2, 2)))
    xq = (xp.reshape(B, 9, 4, 32).transpose(0, 2, 1, 3)).astype(jnp.bfloat16)

    w1s = _conv1_slab(w1)
    w2s = _conv2_slab(w2)
    b1t = jnp.tile(b1.reshape(16), (14,)).reshape(1, 224)
    b2t = jnp.repeat(b2.reshape(32), 7).reshape(1, 224)
    wl = (w_out.reshape(32, 7, 7, 128).transpose(1, 0, 2, 3)
          .reshape(1568, 128).astype(jnp.bfloat16))

    flops = 2 * B * 7 * (288 * 1792 + 1568 * 896) + 2 * B * 1568 * 128
    bytes_accessed = 2 * B * 4 * 9 * 32 + 4 * B * (128 + 1568)

    pf = jnp.asarray(_PF, jnp.bfloat16)

    logits_pad, flat = pl.pallas_call(
        functools.partial(_fused_kernel, tb=tb),
        out_shape=[jax.ShapeDtypeStruct((B, 128), jnp.float32),
                   jax.ShapeDtypeStruct((B, 1568), jnp.float32)],
        grid=(B // tb,),
        in_specs=[pl.BlockSpec((tb, 4, 9, 32), lambda i: (i, 0, 0, 0)),
                  pl.BlockSpec((288, 1792), lambda i: (0, 0)),
                  pl.BlockSpec((1, 224), lambda i: (0, 0)),
                  pl.BlockSpec((1568, 896), lambda i: (0, 0)),
                  pl.BlockSpec((1, 224), lambda i: (0, 0)),
                  pl.BlockSpec((1568, 128), lambda i: (0, 0)),
                  pl.BlockSpec((1, 128), lambda i: (0, 0)),
                  pl.BlockSpec((1568, 1568), lambda i: (0, 0))],
        out_specs=[pl.BlockSpec((tb, 128), lambda i: (i, 0)),
                   pl.BlockSpec((tb, 1568), lambda i: (i, 0))],
        compiler_params=pltpu.CompilerParams(
            dimension_semantics=("parallel",)),
        cost_estimate=pl.CostEstimate(flops=flops, transcendentals=0,
                                      bytes_accessed=bytes_accessed),
    )(xq, w1s, b1t, w2s, b2t, wl, b_out.astype(jnp.float32), pf)

    return logits_pad[:, :10], flat


def kernel(x_nchw, w1, b1, w2, b2, w_out, b_out):
    return _forward(x_nchw, w1, b1, w2, b2, w_out, b_out)
